# baseline (speedup 1.0000x reference)
import functools

import jax
import jax.numpy as jnp
from jax import lax
from jax.experimental import pallas as pl
from jax.experimental.pallas import tpu as pltpu
from jax.experimental.pallas import tpu_sc as plsc

RCUT = 6.0
RCUT_SMTH = 0.5
SEL0 = 46
SEL1 = 92
NNEI = SEL0 + SEL1
NLOC = 8192
NALL = 10240
NOUT = 100

NW = 32
ATOMS_PER_W = NLOC // NW
CHUNK = ATOMS_PER_W * NNEI
SUB = 4416
NSUB = CHUNK // SUB
VEC = 16

ENV_BLK = 512
MLP_BLK = 128


def _sc_gather(coordf, nlistf):
    mesh = plsc.VectorSubcoreMesh(core_axis_name="c", subcore_axis_name="s")

    @functools.partial(
        pl.kernel,
        mesh=mesh,
        out_type=[jax.ShapeDtypeStruct((NLOC * NNEI,), jnp.float32)] * 3,
        compiler_params=pltpu.CompilerParams(needs_layout_passes=False),
        scratch_types=[
            pltpu.VMEM((NALL * 3,), jnp.float32),
            pltpu.VMEM((SUB,), jnp.int32),
            pltpu.VMEM((SUB,), jnp.float32),
            pltpu.VMEM((SUB,), jnp.float32),
            pltpu.VMEM((SUB,), jnp.float32),
        ],
    )
    def k(coord_hbm, nlist_hbm, outx, outy, outz, coord_v, nl_v, xv, yv, zv):
        wid = lax.axis_index("c") * 16 + lax.axis_index("s")
        pltpu.sync_copy(coord_hbm, coord_v)
        base = wid * CHUNK
        for b in range(NSUB):
            off = base + b * SUB
            pltpu.sync_copy(nlist_hbm.at[pl.ds(off, SUB)], nl_v)

            def body(i, carry):
                sl = pl.ds(i * VEC, VEC)
                nl3 = nl_v[sl] * 3
                xv[sl] = plsc.load_gather(coord_v, [nl3])
                yv[sl] = plsc.load_gather(coord_v, [nl3 + 1])
                zv[sl] = plsc.load_gather(coord_v, [nl3 + 2])
                return carry

            lax.fori_loop(0, SUB // VEC, body, 0)
            pltpu.sync_copy(xv, outx.at[pl.ds(off, SUB)])
            pltpu.sync_copy(yv, outy.at[pl.ds(off, SUB)])
            pltpu.sync_copy(zv, outz.at[pl.ds(off, SUB)])

    return k(coordf, nlistf)


def _env_body(x_ref, y_ref, z_ref, cl_ref, at_ref, mean_ref, std_ref,
              o0_ref, o1_ref):
    dx = x_ref[...] - cl_ref[:, 0:1]
    dy = y_ref[...] - cl_ref[:, 1:2]
    dz = z_ref[...] - cl_ref[:, 2:3]
    r2 = dx * dx + dy * dy + dz * dz
    length = jnp.sqrt(r2)
    t0 = 1.0 / length
    min_mask = (length <= RCUT_SMTH).astype(jnp.float32)
    max_mask = (length >= RCUT).astype(jnp.float32)
    mid_mask = 1.0 - jnp.clip(min_mask + max_mask, 0.0, 1.0)
    uu = (length - RCUT_SMTH) / (RCUT - RCUT_SMTH)
    vv = uu * uu * uu * (-6.0 * uu * uu + 15.0 * uu - 10.0) + 1.0
    sw = vv * mid_mask + min_mask
    env = t0 * sw
    sel = at_ref[...] == 0
    m = jnp.where(sel, mean_ref[0:1, :], mean_ref[1:2, :])
    sd = jnp.where(sel, std_ref[0:1, :], std_ref[1:2, :])
    dm = (env - m) / sd
    o0_ref[...] = dm[:, :SEL0]
    o1_ref[...] = dm[:, SEL0:]


def _env(x, y, z, coord_l, atype, mean2, std2):
    grid = NLOC // ENV_BLK
    return pl.pallas_call(
        _env_body,
        grid=(grid,),
        in_specs=[
            pl.BlockSpec((ENV_BLK, NNEI), lambda i: (i, 0)),
            pl.BlockSpec((ENV_BLK, NNEI), lambda i: (i, 0)),
            pl.BlockSpec((ENV_BLK, NNEI), lambda i: (i, 0)),
            pl.BlockSpec((ENV_BLK, 3), lambda i: (i, 0)),
            pl.BlockSpec((ENV_BLK, 1), lambda i: (i, 0)),
            pl.BlockSpec((2, NNEI), lambda i: (0, 0)),
            pl.BlockSpec((2, NNEI), lambda i: (0, 0)),
        ],
        out_specs=[
            pl.BlockSpec((ENV_BLK, SEL0), lambda i: (i, 0)),
            pl.BlockSpec((ENV_BLK, SEL1), lambda i: (i, 0)),
        ],
        out_shape=[
            jax.ShapeDtypeStruct((NLOC, SEL0), jnp.float32),
            jax.ShapeDtypeStruct((NLOC, SEL1), jnp.float32),
        ],
    )(x, y, z, coord_l, atype, mean2, std2)


def _tree_sum(chunks):
    while len(chunks) > 1:
        nxt = [a + b for a, b in zip(chunks[::2], chunks[1::2])]
        if len(chunks) % 2:
            nxt.append(chunks[-1])
        chunks = nxt
    return chunks[0]


def _dot(a, b):
    return jax.lax.dot_general(a, b, (((1,), (0,)), ((), ())),
                               preferred_element_type=jnp.float32)


def _net_sums(xt, nsel, w0c, b0c, w1t, b1c, w2cat, b2c):
    t1 = jnp.tanh(_dot(w0c, xt) + b0c)
    t2 = jnp.tanh(_dot(w1t, t1) + b1c)
    u = jnp.concatenate([t2, t1], axis=0)
    t3 = jnp.tanh(_dot(w2cat, u) + b2c)
    s1 = _tree_sum([t1[:, j * MLP_BLK:(j + 1) * MLP_BLK] for j in range(nsel)])
    s2 = _tree_sum([t2[:, j * MLP_BLK:(j + 1) * MLP_BLK] for j in range(nsel)])
    s3 = _tree_sum([t3[:, j * MLP_BLK:(j + 1) * MLP_BLK] for j in range(nsel)])
    return s1, s2, s3


def _mlp_body(v0_ref, v1_ref,
              w00_ref, b00_ref, w01_ref, b01_ref, w02c_ref, b02_ref,
              w10_ref, b10_ref, w11_ref, b11_ref, w12c_ref, b12_ref,
              out_ref):
    a1, a2, a3 = _net_sums(v0_ref[...], SEL0, w00_ref[...], b00_ref[...],
                           w01_ref[...], b01_ref[...], w02c_ref[...],
                           b02_ref[...])
    c1, c2, c3 = _net_sums(v1_ref[...], SEL1, w10_ref[...], b10_ref[...],
                           w11_ref[...], b11_ref[...], w12c_ref[...],
                           b12_ref[...])
    s1 = a1 + c1
    s2 = a2 + c2
    s3 = a3 + c3
    out = (s3 + jnp.concatenate([s2, s2], axis=0)
           + jnp.concatenate([s1, s1, s1, s1], axis=0))
    out_ref[...] = jnp.transpose(out * (1.0 / (NNEI * 5.0)), (1, 0))


def _mlp(v0t, v1t, ws):
    grid = NLOC // MLP_BLK
    w_specs = []
    for w_or_b in ws:
        shp = w_or_b.shape
        w_specs.append(pl.BlockSpec(shp, lambda i: (0, 0)))
    return pl.pallas_call(
        _mlp_body,
        grid=(grid,),
        in_specs=[
            pl.BlockSpec((1, MLP_BLK * SEL0), lambda i: (0, i)),
            pl.BlockSpec((1, MLP_BLK * SEL1), lambda i: (0, i)),
        ] + w_specs,
        out_specs=pl.BlockSpec((MLP_BLK, NOUT), lambda i: (i, 0)),
        out_shape=jax.ShapeDtypeStruct((NLOC, NOUT), jnp.float32),
    )(v0t, v1t, *ws)


def kernel(coord_ext, atype_ext, nlist, mean, stddev,
           w00, b00, w01, b01, w02, b02, w10, b10, w11, b11, w12, b12):
    coordf = coord_ext.reshape(NALL * 3)
    nlistf = nlist.reshape(NLOC * NNEI)
    x, y, z = _sc_gather(coordf, nlistf)
    x = x.reshape(NLOC, NNEI)
    y = y.reshape(NLOC, NNEI)
    z = z.reshape(NLOC, NNEI)
    coord_l = coord_ext.reshape(NALL, 3)[:NLOC]
    atype = atype_ext.reshape(NALL, 1)[:NLOC]
    mean2 = mean.reshape(2, NNEI)
    std2 = stddev.reshape(2, NNEI)
    dm0, dm1 = _env(x, y, z, coord_l, atype, mean2, std2)
    nb = NLOC // MLP_BLK
    v0t = dm0.reshape(nb, MLP_BLK, SEL0).transpose(0, 2, 1).reshape(1, NLOC * SEL0)
    v1t = dm1.reshape(nb, MLP_BLK, SEL1).transpose(0, 2, 1).reshape(1, NLOC * SEL1)
    w02c = jnp.concatenate([w02.T, (w02[:25, :] + w02[25:, :]).T], axis=1)
    w12c = jnp.concatenate([w12.T, (w12[:25, :] + w12[25:, :]).T], axis=1)
    ws = [w00.T, b00.reshape(-1, 1), w01.T, b01.reshape(-1, 1),
          w02c, b02.reshape(-1, 1),
          w10.T, b10.reshape(-1, 1), w11.T, b11.reshape(-1, 1),
          w12c, b12.reshape(-1, 1)]
    out = _mlp(v0t, v1t, ws)
    return out.reshape(1, NLOC, NOUT)


# speedup vs baseline: 1.0267x; 1.0267x over previous
import functools

import jax
import jax.numpy as jnp
from jax import lax
from jax.experimental import pallas as pl
from jax.experimental.pallas import tpu as pltpu
from jax.experimental.pallas import tpu_sc as plsc

RCUT = 6.0
RCUT_SMTH = 0.5
SEL0 = 46
SEL1 = 92
NNEI = SEL0 + SEL1
NLOC = 8192
NALL = 10240
NOUT = 100

NW = 32
ATOMS_PER_W = NLOC // NW
CHUNK = ATOMS_PER_W * NNEI
SUB = 4416
NSUB = CHUNK // SUB
VEC = 16

ENV_BLK = 512
MLP_BLK = 128


def _sc_gather(coordf, nlistf):
    mesh = plsc.VectorSubcoreMesh(core_axis_name="c", subcore_axis_name="s")

    @functools.partial(
        pl.kernel,
        mesh=mesh,
        out_type=[jax.ShapeDtypeStruct((NLOC * NNEI,), jnp.float32)] * 3,
        compiler_params=pltpu.CompilerParams(needs_layout_passes=False),
        scratch_types=[
            pltpu.VMEM((NALL * 3,), jnp.float32),
            pltpu.VMEM((SUB,), jnp.int32),
            pltpu.VMEM((SUB,), jnp.float32),
            pltpu.VMEM((SUB,), jnp.float32),
            pltpu.VMEM((SUB,), jnp.float32),
        ],
    )
    def k(coord_hbm, nlist_hbm, outx, outy, outz, coord_v, nl_v, xv, yv, zv):
        wid = lax.axis_index("c") * 16 + lax.axis_index("s")
        pltpu.sync_copy(coord_hbm, coord_v)
        base = wid * CHUNK
        for b in range(NSUB):
            off = base + b * SUB
            pltpu.sync_copy(nlist_hbm.at[pl.ds(off, SUB)], nl_v)

            def body(i, carry):
                sl = pl.ds(i * VEC, VEC)
                nl3 = nl_v[sl] * 3
                xv[sl] = plsc.load_gather(coord_v, [nl3])
                yv[sl] = plsc.load_gather(coord_v, [nl3 + 1])
                zv[sl] = plsc.load_gather(coord_v, [nl3 + 2])
                return carry

            lax.fori_loop(0, SUB // VEC, body, 0)
            pltpu.sync_copy(xv, outx.at[pl.ds(off, SUB)])
            pltpu.sync_copy(yv, outy.at[pl.ds(off, SUB)])
            pltpu.sync_copy(zv, outz.at[pl.ds(off, SUB)])

    return k(coordf, nlistf)


def _env_body(x_ref, y_ref, z_ref, cl_ref, at_ref, mean_ref, std_ref,
              o0_ref, o1_ref):
    dx = x_ref[...] - cl_ref[:, 0:1]
    dy = y_ref[...] - cl_ref[:, 1:2]
    dz = z_ref[...] - cl_ref[:, 2:3]
    r2 = dx * dx + dy * dy + dz * dz
    length = jnp.sqrt(r2)
    t0 = 1.0 / length
    min_mask = (length <= RCUT_SMTH).astype(jnp.float32)
    max_mask = (length >= RCUT).astype(jnp.float32)
    mid_mask = 1.0 - jnp.clip(min_mask + max_mask, 0.0, 1.0)
    uu = (length - RCUT_SMTH) / (RCUT - RCUT_SMTH)
    vv = uu * uu * uu * (-6.0 * uu * uu + 15.0 * uu - 10.0) + 1.0
    sw = vv * mid_mask + min_mask
    env = t0 * sw
    sel = at_ref[...] == 0
    m = jnp.where(sel, mean_ref[0:1, :], mean_ref[1:2, :])
    sd = jnp.where(sel, std_ref[0:1, :], std_ref[1:2, :])
    dm = (env - m) / sd
    o0_ref[...] = dm[:, :SEL0]
    o1_ref[...] = dm[:, SEL0:]


def _env(x, y, z, coord_l, atype, mean2, std2):
    grid = NLOC // ENV_BLK
    return pl.pallas_call(
        _env_body,
        grid=(grid,),
        in_specs=[
            pl.BlockSpec((ENV_BLK, NNEI), lambda i: (i, 0)),
            pl.BlockSpec((ENV_BLK, NNEI), lambda i: (i, 0)),
            pl.BlockSpec((ENV_BLK, NNEI), lambda i: (i, 0)),
            pl.BlockSpec((ENV_BLK, 3), lambda i: (i, 0)),
            pl.BlockSpec((ENV_BLK, 1), lambda i: (i, 0)),
            pl.BlockSpec((2, NNEI), lambda i: (0, 0)),
            pl.BlockSpec((2, NNEI), lambda i: (0, 0)),
        ],
        out_specs=[
            pl.BlockSpec((ENV_BLK, SEL0), lambda i: (i, 0)),
            pl.BlockSpec((ENV_BLK, SEL1), lambda i: (i, 0)),
        ],
        out_shape=[
            jax.ShapeDtypeStruct((NLOC, SEL0), jnp.float32),
            jax.ShapeDtypeStruct((NLOC, SEL1), jnp.float32),
        ],
    )(x, y, z, coord_l, atype, mean2, std2)


def _tree_sum(chunks):
    while len(chunks) > 1:
        nxt = [a + b for a, b in zip(chunks[::2], chunks[1::2])]
        if len(chunks) % 2:
            nxt.append(chunks[-1])
        chunks = nxt
    return chunks[0]


def _dot(a, b):
    return jax.lax.dot_general(a, b, (((1,), (0,)), ((), ())),
                               preferred_element_type=jnp.float32)


def _net_sums(xt, nsel, w0c, b0c, w1t, b1c, w2cat, b2c):
    t1 = jnp.tanh(_dot(w0c, xt) + b0c)
    t2 = jnp.tanh(_dot(w1t, t1) + b1c)
    u = jnp.concatenate([t2, t1], axis=0)
    t3 = jnp.tanh(_dot(w2cat, u) + b2c)
    s1 = _tree_sum([t1[:, j * MLP_BLK:(j + 1) * MLP_BLK] for j in range(nsel)])
    s2 = _tree_sum([t2[:, j * MLP_BLK:(j + 1) * MLP_BLK] for j in range(nsel)])
    s3 = _tree_sum([t3[:, j * MLP_BLK:(j + 1) * MLP_BLK] for j in range(nsel)])
    return s1, s2, s3


def _mlp_body(v0_ref, v1_ref,
              w00_ref, b00_ref, w01_ref, b01_ref, w02c_ref, b02_ref,
              w10_ref, b10_ref, w11_ref, b11_ref, w12c_ref, b12_ref,
              out_ref):
    a1, a2, a3 = _net_sums(v0_ref[...], SEL0, w00_ref[...], b00_ref[...],
                           w01_ref[...], b01_ref[...], w02c_ref[...],
                           b02_ref[...])
    c1, c2, c3 = _net_sums(v1_ref[...], SEL1, w10_ref[...], b10_ref[...],
                           w11_ref[...], b11_ref[...], w12c_ref[...],
                           b12_ref[...])
    s1 = a1 + c1
    s2 = a2 + c2
    s3 = a3 + c3
    out = (s3 + jnp.concatenate([s2, s2], axis=0)
           + jnp.concatenate([s1, s1, s1, s1], axis=0))
    out_ref[...] = out * (1.0 / (NNEI * 5.0))


def _mlp(v0t, v1t, ws):
    grid = NLOC // MLP_BLK
    w_specs = []
    for w_or_b in ws:
        shp = w_or_b.shape
        w_specs.append(pl.BlockSpec(shp, lambda i: (0, 0)))
    return pl.pallas_call(
        _mlp_body,
        grid=(grid,),
        in_specs=[
            pl.BlockSpec((1, MLP_BLK * SEL0), lambda i: (0, i)),
            pl.BlockSpec((1, MLP_BLK * SEL1), lambda i: (0, i)),
        ] + w_specs,
        out_specs=pl.BlockSpec((NOUT, MLP_BLK), lambda i: (0, i)),
        out_shape=jax.ShapeDtypeStruct((NOUT, NLOC), jnp.float32),
    )(v0t, v1t, *ws)


def kernel(coord_ext, atype_ext, nlist, mean, stddev,
           w00, b00, w01, b01, w02, b02, w10, b10, w11, b11, w12, b12):
    coordf = coord_ext.reshape(NALL * 3)
    nlistf = nlist.reshape(NLOC * NNEI)
    x, y, z = _sc_gather(coordf, nlistf)
    x = x.reshape(NLOC, NNEI)
    y = y.reshape(NLOC, NNEI)
    z = z.reshape(NLOC, NNEI)
    coord_l = coord_ext.reshape(NALL, 3)[:NLOC]
    atype = atype_ext.reshape(NALL, 1)[:NLOC]
    mean2 = mean.reshape(2, NNEI)
    std2 = stddev.reshape(2, NNEI)
    dm0, dm1 = _env(x, y, z, coord_l, atype, mean2, std2)
    nb = NLOC // MLP_BLK
    v0t = dm0.reshape(nb, MLP_BLK, SEL0).transpose(0, 2, 1).reshape(1, NLOC * SEL0)
    v1t = dm1.reshape(nb, MLP_BLK, SEL1).transpose(0, 2, 1).reshape(1, NLOC * SEL1)
    w02c = jnp.concatenate([w02.T, (w02[:25, :] + w02[25:, :]).T], axis=1)
    w12c = jnp.concatenate([w12.T, (w12[:25, :] + w12[25:, :]).T], axis=1)
    ws = [w00.T, b00.reshape(-1, 1), w01.T, b01.reshape(-1, 1),
          w02c, b02.reshape(-1, 1),
          w10.T, b10.reshape(-1, 1), w11.T, b11.reshape(-1, 1),
          w12c, b12.reshape(-1, 1)]
    out_t = _mlp(v0t, v1t, ws)
    return out_t.T.reshape(1, NLOC, NOUT)
